# baseline (speedup 1.0000x reference)
import functools

import jax
import jax.numpy as jnp
from jax import lax
from jax.experimental import pallas as pl
from jax.experimental.pallas import tpu as pltpu
from jax.experimental.pallas import tpu_sc as plsc

_INFO = plsc.get_sparse_core_info()
_NC, _NS, _L = _INFO.num_cores, _INFO.num_subcores, _INFO.num_lanes
_NW = _NC * _NS

_NCLS = 91
_NR = 1000
_N_PAD = 32768
_BLK = 128
_ROWS_C = (768, 1280)
_ROWS_MAX = max(_ROWS_C)
_SEG = sum(_ROWS_C)


def _tile_work(nrows, base, wid,
               tci_hbm, tbt_hbm, pred_hbm, out_hbm,
               tci_v, idx_v, rows_v, tb_v, acc_v, cnt_v,
               tb_sem, sems):
    nblk = nrows // _BLK

    pltpu.sync_copy(tci_hbm.at[pl.ds(base, nrows)], tci_v.at[pl.ds(0, nrows)])
    tb_copy = pltpu.async_copy(
        tbt_hbm.at[:, pl.ds(base, nrows)], tb_v.at[:, pl.ds(0, nrows)],
        tb_sem,
    )

    lane = lax.iota(jnp.int32, _L)

    copies = []
    for g in range(nblk):
        for s in range(_BLK // _L):
            off = g * _BLK + s * _L
            v = tci_v[pl.ds(off, _L)]
            pos = v > 0
            roi = base + off + lane
            b = lax.shift_right_logical(
                roi.astype(jnp.uint32) * jnp.uint32(67109), jnp.uint32(26)
            ).astype(jnp.int32)
            rr = roi - b * _NR
            b16 = (b * _NCLS + v) * 4
            for c in range(4):
                idx_v[pl.ds((g * 4 + c) * _BLK + s * _L, _L)] = jnp.where(
                    pos, (b16 + c) * _NR + rr, 0
                )
        copies.append(pltpu.async_copy(
            pred_hbm.at[idx_v.at[pl.ds(g * 4 * _BLK, 4 * _BLK)]],
            rows_v.at[pl.ds(g * 4 * _BLK, 4 * _BLK)],
            sems[g],
        ))

    tb_copy.wait()

    zero = jnp.zeros((_L,), jnp.float32)
    acc, cnt = zero, zero
    for g in range(nblk):
        copies[g].wait()

        def step(k, carry, g=g):
            acc, cnt = carry
            o = k * _L
            cls16 = tci_v[pl.ds(g * _BLK + o, _L)]
            posf = jnp.where(cls16 > 0, 1.0, 0.0).astype(jnp.float32)
            cnt = cnt + posf
            for c in range(4):
                pred16 = rows_v[pl.ds((g * 4 + c) * _BLK + o, _L)]
                tb16 = tb_v[c, pl.ds(g * _BLK + o, _L)]
                diff = jnp.abs(tb16 - pred16)
                sl1 = jnp.where(diff < 1.0, 0.5 * diff * diff, diff - 0.5)
                acc = acc + sl1 * posf
            return acc, cnt

        acc, cnt = lax.fori_loop(0, _BLK // _L, step, (acc, cnt))

    acc_v[...] = acc
    cnt_v[...] = cnt
    pltpu.sync_copy(acc_v, out_hbm.at[wid, 0])
    pltpu.sync_copy(cnt_v, out_hbm.at[wid, 1])


def _sc_body(tci_hbm, tbt_hbm, pred_hbm, out_hbm,
             tci_v, idx_v, rows_v, tb_v, acc_v, cnt_v,
             tb_sem, *sems):
    sidx = lax.axis_index("s")
    cidx = lax.axis_index("c")
    wid = sidx * _NC + cidx
    refs = (tci_hbm, tbt_hbm, pred_hbm, out_hbm,
            tci_v, idx_v, rows_v, tb_v, acc_v, cnt_v, tb_sem, sems)

    @pl.when(cidx == 0)
    def _():
        _tile_work(_ROWS_C[0], sidx * _SEG, wid, *refs)

    @pl.when(cidx == 1)
    def _():
        _tile_work(_ROWS_C[1], sidx * _SEG + _ROWS_C[0], wid, *refs)


@functools.partial(
    pl.kernel,
    out_type=jax.ShapeDtypeStruct((_NW, 2, _L), jnp.float32),
    scratch_types=[
        pltpu.VMEM((_ROWS_MAX,), jnp.int32),
        pltpu.VMEM((_ROWS_MAX * 4,), jnp.int32),
        pltpu.VMEM((_ROWS_MAX * 4,), jnp.float32),
        pltpu.VMEM((4, _ROWS_MAX), jnp.float32),
        pltpu.VMEM((_L,), jnp.float32),
        pltpu.VMEM((_L,), jnp.float32),
        pltpu.SemaphoreType.DMA,
    ] + [pltpu.SemaphoreType.DMA] * (_ROWS_MAX // _BLK),
    mesh=plsc.VectorSubcoreMesh(core_axis_name="c", subcore_axis_name="s"),
)
def _sc_loss(tci_hbm, tbt_hbm, pred_hbm, out_hbm, *scratch):
    _sc_body(tci_hbm, tbt_hbm, pred_hbm, out_hbm, *scratch)


def kernel(target_bbox, target_class_ids, pred_bbox):
    n = target_class_ids.shape[0] * target_class_ids.shape[1]
    tci = target_class_ids.reshape(-1).astype(jnp.int32)
    tci = jnp.pad(tci, (0, _N_PAD - n))
    tbt = jnp.pad(target_bbox.reshape(-1, 4).T, ((0, 0), (0, _N_PAD - n)))
    pred_flat = jnp.transpose(pred_bbox, (0, 2, 3, 1)).reshape(-1)
    parts = _sc_loss(tci, tbt, pred_flat)
    total = jnp.sum(parts[:, 0, :])
    count = jnp.sum(parts[:, 1, :])
    return total / (count * 4.0)


# speedup vs baseline: 1.0170x; 1.0170x over previous
import functools

import jax
import jax.numpy as jnp
from jax import lax
from jax.experimental import pallas as pl
from jax.experimental.pallas import tpu as pltpu
from jax.experimental.pallas import tpu_sc as plsc

_INFO = plsc.get_sparse_core_info()
_NC, _NS, _L = _INFO.num_cores, _INFO.num_subcores, _INFO.num_lanes
_NW = _NC * _NS

_NCLS = 91
_NR = 1000
_N_PAD = 32768
_ROWS_PER_W = _N_PAD // _NW
_BLK = 128
_NBLK = _ROWS_PER_W // _BLK


def _sc_body(tci_hbm, tbt_hbm, pred_hbm, out_hbm,
             tci_v, idx_v, rows_v, tb_v, acc_v, cnt_v,
             tb_sem, *sems):
    wid = lax.axis_index("s") * _NC + lax.axis_index("c")
    base = wid * _ROWS_PER_W

    pltpu.sync_copy(tci_hbm.at[pl.ds(base, _ROWS_PER_W)], tci_v)
    tb_copy = pltpu.async_copy(
        tbt_hbm.at[:, pl.ds(base, _ROWS_PER_W)], tb_v, tb_sem
    )

    lane = lax.iota(jnp.int32, _L)

    copies = []
    for g in range(_NBLK):
        for s in range(_BLK // _L):
            off = g * _BLK + s * _L
            v = tci_v[pl.ds(off, _L)]
            pos = v > 0
            roi = base + off + lane
            b = lax.shift_right_logical(
                roi.astype(jnp.uint32) * jnp.uint32(67109), jnp.uint32(26)
            ).astype(jnp.int32)
            rr = roi - b * _NR
            b16 = (b * _NCLS + v) * 4
            for c in range(4):
                idx_v[pl.ds((g * 4 + c) * _BLK + s * _L, _L)] = jnp.where(
                    pos, (b16 + c) * _NR + rr, 0
                )
        copies.append(pltpu.async_copy(
            pred_hbm.at[idx_v.at[pl.ds(g * 4 * _BLK, 4 * _BLK)]],
            rows_v.at[pl.ds(g * 4 * _BLK, 4 * _BLK)],
            sems[g],
        ))

    tb_copy.wait()

    zero = jnp.zeros((_L,), jnp.float32)
    acc, cnt = zero, zero
    for g in range(_NBLK):
        copies[g].wait()

        def step(k, carry, g=g):
            acc, cnt = carry
            o = k * _L
            cls16 = tci_v[pl.ds(g * _BLK + o, _L)]
            posf = jnp.where(cls16 > 0, 1.0, 0.0).astype(jnp.float32)
            cnt = cnt + posf
            for c in range(4):
                pred16 = rows_v[pl.ds((g * 4 + c) * _BLK + o, _L)]
                tb16 = tb_v[c, pl.ds(g * _BLK + o, _L)]
                diff = jnp.abs(tb16 - pred16)
                sl1 = jnp.where(diff < 1.0, 0.5 * diff * diff, diff - 0.5)
                acc = acc + sl1 * posf
            return acc, cnt

        acc, cnt = lax.fori_loop(0, _BLK // _L, step, (acc, cnt))

    acc_v[...] = acc
    cnt_v[...] = cnt
    pltpu.sync_copy(acc_v, out_hbm.at[wid, 0])
    pltpu.sync_copy(cnt_v, out_hbm.at[wid, 1])


@functools.partial(
    pl.kernel,
    out_type=jax.ShapeDtypeStruct((_NW, 2, _L), jnp.float32),
    scratch_types=[
        pltpu.VMEM((_ROWS_PER_W,), jnp.int32),
        pltpu.VMEM((_ROWS_PER_W * 4,), jnp.int32),
        pltpu.VMEM((_ROWS_PER_W * 4,), jnp.float32),
        pltpu.VMEM((4, _ROWS_PER_W), jnp.float32),
        pltpu.VMEM((_L,), jnp.float32),
        pltpu.VMEM((_L,), jnp.float32),
        pltpu.SemaphoreType.DMA,
    ] + [pltpu.SemaphoreType.DMA] * _NBLK,
    mesh=plsc.VectorSubcoreMesh(core_axis_name="c", subcore_axis_name="s"),
)
def _sc_loss(tci_hbm, tbt_hbm, pred_hbm, out_hbm, *scratch):
    _sc_body(tci_hbm, tbt_hbm, pred_hbm, out_hbm, *scratch)


def kernel(target_bbox, target_class_ids, pred_bbox):
    n = target_class_ids.shape[0] * target_class_ids.shape[1]
    tci = target_class_ids.reshape(-1).astype(jnp.int32)
    tci = jnp.pad(tci, (0, _N_PAD - n))
    tbt = jnp.pad(target_bbox.reshape(-1, 4).T, ((0, 0), (0, _N_PAD - n)))
    pred_flat = jnp.transpose(pred_bbox, (0, 2, 3, 1)).reshape(-1)
    parts = _sc_loss(tci, tbt, pred_flat)
    total = jnp.sum(parts[:, 0, :])
    count = jnp.sum(parts[:, 1, :])
    return total / (count * 4.0)
